# baseline (speedup 1.0000x reference)
import functools

import jax
import jax.numpy as jnp
from jax import lax
from jax.experimental import pallas as pl
from jax.experimental.pallas import tpu as pltpu
from jax.experimental.pallas import tpu_sc as plsc

_EPS = 1e-12
_LANES = 16
_NUM_WORKERS = 32
_CHUNK = 32


@functools.lru_cache(maxsize=None)
def _build(batch: int, seq: int, hidden: int):
    n_rows = batch * seq
    spw = seq // _NUM_WORKERS
    assert seq % _NUM_WORKERS == 0 and spw % _CHUNK == 0
    n_sub = (batch * spw) // _CHUNK
    chunks_per_b = spw // _CHUNK
    n_vec = hidden // _LANES
    assert hidden % _LANES == 0

    mesh = plsc.VectorSubcoreMesh(core_axis_name="core", subcore_axis_name="sub")

    @functools.partial(
        pl.kernel,
        mesh=mesh,
        out_type=jax.ShapeDtypeStruct((n_rows, hidden), jnp.float32),
        scratch_types=[
            pltpu.VMEM((n_sub, _CHUNK), jnp.int32),
            pltpu.VMEM((spw, hidden), jnp.float32),
            pltpu.VMEM((_CHUNK, hidden), jnp.float32),
            pltpu.VMEM((hidden,), jnp.float32),
            pltpu.VMEM((hidden,), jnp.float32),
            pltpu.SemaphoreType.DMA,
        ],
    )
    def embed_ln(ids_hbm, wt_hbm, pt_hbm, g_hbm, b_hbm, out_hbm,
                 idx_v, pos_v, row_v, gam_v, bet_v, sem):
        wid = lax.axis_index("sub") * 2 + lax.axis_index("core")
        s0 = wid * spw
        pltpu.sync_copy(g_hbm, gam_v)
        pltpu.sync_copy(b_hbm, bet_v)
        pltpu.sync_copy(pt_hbm.at[pl.ds(s0, spw)], pos_v)

        def norm_rows(pos_off):
            def row_body(r, _):
                def acc(j, carry):
                    s, s2 = carry
                    col = pl.ds(j * _LANES, _LANES)
                    x = row_v[r, col] + pos_v[pos_off + r, col]
                    row_v[r, col] = x
                    return s + x, s2 + x * x

                z = jnp.zeros((_LANES,), jnp.float32)
                s, s2 = lax.fori_loop(0, n_vec, acc, (z, z))
                lane = lax.iota(jnp.int32, _LANES)
                for step in (8, 4, 2, 1):
                    perm = lane ^ step
                    s = s + s.at[perm].get(mode="promise_in_bounds")
                    s2 = s2 + s2.at[perm].get(mode="promise_in_bounds")
                mean = s * (1.0 / hidden)
                var = s2 * (1.0 / hidden) - mean * mean
                v0 = (var[0] + _EPS).astype(jnp.float32)
                iv = lax.bitcast_convert_type(v0, jnp.int32)
                y0 = lax.bitcast_convert_type(
                    jnp.int32(0x5F3759DF) - (iv >> 1), jnp.float32)
                for _unused in range(3):
                    y0 = y0 * (1.5 - 0.5 * v0 * y0 * y0)
                y = jnp.full((_LANES,), y0, jnp.float32)
                m = mean

                def nrm(j, _):
                    col = pl.ds(j * _LANES, _LANES)
                    x = row_v[r, col]
                    row_v[r, col] = (x - m) * y * gam_v[col] + bet_v[col]
                    return 0

                lax.fori_loop(0, n_vec, nrm, 0)
                return 0

            lax.fori_loop(0, _CHUNK, row_body, 0)

        for k in range(n_sub):
            b, c = divmod(k, chunks_per_b)
            row0 = b * seq + s0 + c * _CHUNK
            pltpu.sync_copy(ids_hbm.at[pl.ds(row0, _CHUNK)], idx_v.at[k])
            pltpu.async_copy(wt_hbm.at[idx_v.at[k]], row_v, sem).wait()
            norm_rows(c * _CHUNK)
            pltpu.sync_copy(row_v, out_hbm.at[pl.ds(row0, _CHUNK)])

    return embed_ln


def kernel(input_ids, word_table, pos_table, ln_gamma, ln_beta):
    batch, seq = input_ids.shape
    hidden = word_table.shape[1]
    ids = jnp.asarray(input_ids, jnp.int32).reshape(-1)
    fn = _build(batch, seq, hidden)
    out = fn(ids, word_table, pos_table,
             jnp.asarray(ln_gamma, jnp.float32),
             jnp.asarray(ln_beta, jnp.float32))
    return out.reshape(batch, seq, hidden)


# speedup vs baseline: 1.1699x; 1.1699x over previous
import functools

import jax
import jax.numpy as jnp
from jax import lax
from jax.experimental import pallas as pl
from jax.experimental.pallas import tpu as pltpu
from jax.experimental.pallas import tpu_sc as plsc

_EPS = 1e-12
_LANES = 16
_NUM_WORKERS = 32
_CHUNK = 32
_NBUF = 3
_UNROLL = 8


@functools.lru_cache(maxsize=None)
def _build(batch: int, seq: int, hidden: int):
    n_rows = batch * seq
    spw = seq // _NUM_WORKERS
    assert seq % _NUM_WORKERS == 0 and spw % _CHUNK == 0
    n_sub = (batch * spw) // _CHUNK
    chunks_per_b = spw // _CHUNK
    n_vec = hidden // _LANES
    assert hidden % (_LANES * _UNROLL) == 0
    n_outer = n_vec // _UNROLL

    mesh = plsc.VectorSubcoreMesh(core_axis_name="core", subcore_axis_name="sub")

    @functools.partial(
        pl.kernel,
        mesh=mesh,
        out_type=jax.ShapeDtypeStruct((n_rows, hidden), jnp.float32),
        scratch_types=[
            pltpu.VMEM((n_sub, _CHUNK), jnp.int32),
            pltpu.VMEM((spw, hidden), jnp.float32),
            pltpu.VMEM((_NBUF, _CHUNK, hidden), jnp.float32),
            pltpu.VMEM((hidden,), jnp.float32),
            pltpu.VMEM((hidden,), jnp.float32),
            pltpu.SemaphoreType.DMA,
            pltpu.SemaphoreType.DMA,
            pltpu.SemaphoreType.DMA,
            pltpu.SemaphoreType.DMA,
            pltpu.SemaphoreType.DMA,
            pltpu.SemaphoreType.DMA,
            pltpu.SemaphoreType.DMA,
        ],
    )
    def embed_ln(ids_hbm, wt_hbm, pt_hbm, g_hbm, b_hbm, out_hbm,
                 idx_v, pos_v, row_v, gam_v, bet_v,
                 sem_i, sg0, sg1, sg2, so0, so1, so2):
        sem_g = (sg0, sg1, sg2)
        sem_o = (so0, so1, so2)
        wid = lax.axis_index("sub") * 2 + lax.axis_index("core")
        s0 = wid * spw

        def chunk_row0(k):
            b, c = divmod(k, chunks_per_b)
            return b * seq + s0 + c * _CHUNK

        stage = [pltpu.async_copy(ids_hbm.at[pl.ds(chunk_row0(k), _CHUNK)],
                                  idx_v.at[k], sem_i)
                 for k in range(n_sub)]
        stage.append(pltpu.async_copy(g_hbm, gam_v, sem_i))
        stage.append(pltpu.async_copy(b_hbm, bet_v, sem_i))
        stage.append(pltpu.async_copy(pt_hbm.at[pl.ds(s0, spw)], pos_v, sem_i))
        for h in stage:
            h.wait()

        def norm_rows(buf, pos_off):
            def row_body(r, _):
                z = jnp.zeros((_LANES,), jnp.float32)

                def acc(jo, carry):
                    s_ch = list(carry[0])
                    q_ch = list(carry[1])
                    base = jo * (_UNROLL * _LANES)
                    for u in range(_UNROLL):
                        col = pl.ds(base + u * _LANES, _LANES)
                        x = row_v[buf, r, col] + pos_v[pos_off + r, col]
                        row_v[buf, r, col] = x
                        s_ch[u % 4] = s_ch[u % 4] + x
                        q_ch[u % 4] = q_ch[u % 4] + x * x
                    return tuple(s_ch), tuple(q_ch)

                s_ch, q_ch = lax.fori_loop(0, n_outer, acc,
                                           ((z,) * 4, (z,) * 4))
                s = (s_ch[0] + s_ch[1]) + (s_ch[2] + s_ch[3])
                s2 = (q_ch[0] + q_ch[1]) + (q_ch[2] + q_ch[3])
                lane = lax.iota(jnp.int32, _LANES)
                for step in (8, 4, 2, 1):
                    perm = lane ^ step
                    s = s + s.at[perm].get(mode="promise_in_bounds")
                    s2 = s2 + s2.at[perm].get(mode="promise_in_bounds")
                mean = s * (1.0 / hidden)
                var = s2 * (1.0 / hidden) - mean * mean
                v0 = var[0] + _EPS
                iv = lax.bitcast_convert_type(v0, jnp.int32)
                y0 = lax.bitcast_convert_type(
                    jnp.int32(0x5F3759DF) - (iv >> 1), jnp.float32)
                for _unused in range(3):
                    y0 = y0 * (1.5 - 0.5 * v0 * y0 * y0)
                y = jnp.full((_LANES,), y0, jnp.float32)
                m = mean

                def nrm(jo, _):
                    base = jo * (_UNROLL * _LANES)
                    for u in range(_UNROLL):
                        col = pl.ds(base + u * _LANES, _LANES)
                        x = row_v[buf, r, col]
                        row_v[buf, r, col] = (x - m) * y * gam_v[col] + bet_v[col]
                    return 0

                lax.fori_loop(0, n_outer, nrm, 0)
                return 0

            lax.fori_loop(0, _CHUNK, row_body, 0)

        gather_h = [None] * n_sub
        out_h = [None] * n_sub

        def start_gather(k):
            gather_h[k] = pltpu.async_copy(
                wt_hbm.at[idx_v.at[k]], row_v.at[k % _NBUF], sem_g[k % _NBUF])

        start_gather(0)
        for k in range(n_sub):
            buf = k % _NBUF
            if k + 1 < n_sub:
                if k - 2 >= 0:
                    out_h[k - 2].wait()
                start_gather(k + 1)
            gather_h[k].wait()
            norm_rows(buf, (k % chunks_per_b) * _CHUNK)
            out_h[k] = pltpu.async_copy(
                row_v.at[buf], out_hbm.at[pl.ds(chunk_row0(k), _CHUNK)],
                sem_o[buf])
        for k in range(max(0, n_sub - 2), n_sub):
            out_h[k].wait()
        if n_sub >= 3:
            out_h[n_sub - 3].wait()

    return embed_ln


def kernel(input_ids, word_table, pos_table, ln_gamma, ln_beta):
    batch, seq = input_ids.shape
    hidden = word_table.shape[1]
    ids = jnp.asarray(input_ids, jnp.int32).reshape(-1)
    fn = _build(batch, seq, hidden)
    out = fn(ids, word_table, pos_table,
             jnp.asarray(ln_gamma, jnp.float32),
             jnp.asarray(ln_beta, jnp.float32))
    return out.reshape(batch, seq, hidden)


# speedup vs baseline: 1.6329x; 1.3958x over previous
import functools

import jax
import jax.numpy as jnp
from jax import lax
from jax.experimental import pallas as pl
from jax.experimental.pallas import tpu as pltpu
from jax.experimental.pallas import tpu_sc as plsc

_EPS = 1e-12
_LANES = 16
_NUM_WORKERS = 32
_CHUNK = 32
_NBUF = 3
_UNROLL = 8


@functools.lru_cache(maxsize=None)
def _build(batch: int, seq: int, hidden: int):
    n_rows = batch * seq
    spw = seq // _NUM_WORKERS
    assert seq % _NUM_WORKERS == 0 and spw % _CHUNK == 0
    n_sub = (batch * spw) // _CHUNK
    chunks_per_b = spw // _CHUNK
    n_vec = hidden // _LANES
    assert hidden % (_LANES * _UNROLL) == 0
    n_outer = n_vec // _UNROLL

    mesh = plsc.VectorSubcoreMesh(core_axis_name="core", subcore_axis_name="sub")

    @functools.partial(
        pl.kernel,
        mesh=mesh,
        out_type=jax.ShapeDtypeStruct((n_rows, hidden), jnp.float32),
        scratch_types=[
            pltpu.VMEM((n_sub, _CHUNK), jnp.int32),
            pltpu.VMEM((spw, hidden), jnp.float32),
            pltpu.VMEM((_NBUF, _CHUNK, hidden), jnp.float32),
            pltpu.VMEM((hidden,), jnp.float32),
            pltpu.VMEM((hidden,), jnp.float32),
            pltpu.SemaphoreType.DMA,
            pltpu.SemaphoreType.DMA,
            pltpu.SemaphoreType.DMA,
            pltpu.SemaphoreType.DMA,
            pltpu.SemaphoreType.DMA,
            pltpu.SemaphoreType.DMA,
            pltpu.SemaphoreType.DMA,
        ],
    )
    def embed_ln(ids_hbm, wt_hbm, pt_hbm, g_hbm, b_hbm, out_hbm,
                 idx_v, pos_v, row_v, gam_v, bet_v,
                 sem_i, sg0, sg1, sg2, so0, so1, so2):
        sem_g = (sg0, sg1, sg2)
        sem_o = (so0, so1, so2)
        wid = lax.axis_index("sub") * 2 + lax.axis_index("core")
        s0 = wid * spw

        def chunk_row0(k):
            b, c = divmod(k, chunks_per_b)
            return b * seq + s0 + c * _CHUNK

        stage = [pltpu.async_copy(ids_hbm.at[pl.ds(chunk_row0(k), _CHUNK)],
                                  idx_v.at[k], sem_i)
                 for k in range(n_sub)]
        stage.append(pltpu.async_copy(g_hbm, gam_v, sem_i))
        stage.append(pltpu.async_copy(b_hbm, bet_v, sem_i))
        stage.append(pltpu.async_copy(pt_hbm.at[pl.ds(s0, spw)], pos_v, sem_i))
        for h in stage:
            h.wait()

        def norm_rows(buf, pos_off):
            def row_body(r, _):
                z = jnp.zeros((_LANES,), jnp.float32)

                @plsc.parallel_loop(0, n_outer, carry=((z,) * 4, (z,) * 4))
                def acc(jo, carry):
                    s_ch = list(carry[0])
                    q_ch = list(carry[1])
                    base = jo * (_UNROLL * _LANES)
                    for u in range(_UNROLL):
                        col = pl.ds(base + u * _LANES, _LANES)
                        x = row_v[buf, r, col] + pos_v[pos_off + r, col]
                        row_v[buf, r, col] = x
                        s_ch[u % 4] = s_ch[u % 4] + x
                        q_ch[u % 4] = q_ch[u % 4] + x * x
                    return tuple(s_ch), tuple(q_ch)

                s_ch, q_ch = acc
                s = (s_ch[0] + s_ch[1]) + (s_ch[2] + s_ch[3])
                s2 = (q_ch[0] + q_ch[1]) + (q_ch[2] + q_ch[3])
                lane = lax.iota(jnp.int32, _LANES)
                for step in (8, 4, 2, 1):
                    perm = lane ^ step
                    s = s + s.at[perm].get(mode="promise_in_bounds")
                    s2 = s2 + s2.at[perm].get(mode="promise_in_bounds")
                mean = s * (1.0 / hidden)
                var = s2 * (1.0 / hidden) - mean * mean
                v0 = var[0] + _EPS
                iv = lax.bitcast_convert_type(v0, jnp.int32)
                y0 = lax.bitcast_convert_type(
                    jnp.int32(0x5F3759DF) - (iv >> 1), jnp.float32)
                for _unused in range(3):
                    y0 = y0 * (1.5 - 0.5 * v0 * y0 * y0)
                y = jnp.full((_LANES,), y0, jnp.float32)
                m = mean

                @plsc.parallel_loop(0, n_outer)
                def nrm(jo):
                    base = jo * (_UNROLL * _LANES)
                    for u in range(_UNROLL):
                        col = pl.ds(base + u * _LANES, _LANES)
                        x = row_v[buf, r, col]
                        row_v[buf, r, col] = (x - m) * y * gam_v[col] + bet_v[col]
                return 0

            lax.fori_loop(0, _CHUNK, row_body, 0)

        gather_h = [None] * n_sub
        out_h = [None] * n_sub

        def start_gather(k):
            gather_h[k] = pltpu.async_copy(
                wt_hbm.at[idx_v.at[k]], row_v.at[k % _NBUF], sem_g[k % _NBUF])

        start_gather(0)
        for k in range(n_sub):
            buf = k % _NBUF
            if k + 1 < n_sub:
                if k - 2 >= 0:
                    out_h[k - 2].wait()
                start_gather(k + 1)
            gather_h[k].wait()
            norm_rows(buf, (k % chunks_per_b) * _CHUNK)
            out_h[k] = pltpu.async_copy(
                row_v.at[buf], out_hbm.at[pl.ds(chunk_row0(k), _CHUNK)],
                sem_o[buf])
        for k in range(max(0, n_sub - 2), n_sub):
            out_h[k].wait()
        if n_sub >= 3:
            out_h[n_sub - 3].wait()

    return embed_ln


def kernel(input_ids, word_table, pos_table, ln_gamma, ln_beta):
    batch, seq = input_ids.shape
    hidden = word_table.shape[1]
    ids = jnp.asarray(input_ids, jnp.int32).reshape(-1)
    fn = _build(batch, seq, hidden)
    out = fn(ids, word_table, pos_table,
             jnp.asarray(ln_gamma, jnp.float32),
             jnp.asarray(ln_beta, jnp.float32))
    return out.reshape(batch, seq, hidden)
